# baseline (speedup 1.0000x reference)
import jax
import jax.numpy as jnp
import numpy as np
from jax import lax
from jax.experimental import pallas as pl
from jax.experimental.pallas import tpu as pltpu

LANES = 128
ROWS_PER_BLOCK = 1024


def _brev8(v):
    v = ((v & 0x0F) << 4) | (v >> 4)
    v = ((v & 0x33) << 2) | ((v >> 2) & 0x33)
    v = ((v & 0x55) << 1) | ((v >> 1) & 0x55)
    return v


def _bsadd_block(a_ref, b_ref, o_ref, carry_ref):
    @pl.when(pl.program_id(0) == 0)
    def _():
        carry_ref[0] = 0

    a = _brev8(a_ref[...].astype(jnp.int32))
    b = _brev8(b_ref[...].astype(jnp.int32))
    c = a + b
    g = c >> 8
    c = c & 0xFF
    p = (c == 0xFF)

    R, C = c.shape
    H = C // 16
    two = jnp.int32(2)

    ji = lax.broadcasted_iota(jnp.int32, (C, H), 0)
    hi = lax.broadcasted_iota(jnp.int32, (C, H), 1)
    w_pack = jnp.where(ji // 16 == hi, 1 << (ji % 16), 0).astype(jnp.float32)
    gm = jnp.dot(g.astype(jnp.float32), w_pack,
                 preferred_element_type=jnp.float32).astype(jnp.int32)
    pm = jnp.dot(p.astype(jnp.float32), w_pack,
                 preferred_element_type=jnp.float32).astype(jnp.int32)

    pg = pm | gm
    fh = jnp.where(pm == 0xFFFF, 2, (gm + pg) >> 16)

    f = fh
    k = 1
    while k < H:
        shifted = jnp.concatenate(
            [jnp.full((R, k), two, jnp.int32), f[:, : H - k]], axis=1)
        f = jnp.where(f == two, shifted, f)
        k *= 2

    agg = f[:, H - 1:H]
    k = 1
    while k < R:
        shifted = jnp.concatenate(
            [jnp.full((k, 1), two, jnp.int32), agg[: R - k]], axis=0)
        agg = jnp.where(agg == two, shifted, agg)
        k *= 2

    carry_in = carry_ref[0]
    row_excl = jnp.concatenate(
        [jnp.full((1, 1), two, jnp.int32), agg[: R - 1]], axis=0)
    row_pref = jnp.where(row_excl == two, carry_in, row_excl)

    e = jnp.concatenate(
        [jnp.full((R, 1), two, jnp.int32), f[:, : H - 1]], axis=1)
    ch = jnp.where(e == two, row_pref, e)

    carries = ((gm + pg + ch) ^ gm ^ pg) & 0xFFFF

    car_lo = carries & 0xFF
    car_hi = carries >> 8
    cat = jnp.concatenate([car_lo, car_hi], axis=1).astype(jnp.float32)
    ri = lax.broadcasted_iota(jnp.int32, (2 * H, C), 0)
    jj = lax.broadcasted_iota(jnp.int32, (2 * H, C), 1)
    e_sel = jnp.where(jj // 16 == ri % H,
                      jnp.where(ri < H, 1, 256), 0).astype(jnp.float32)
    v = jnp.dot(cat, e_sel,
                preferred_element_type=jnp.float32).astype(jnp.int32)
    lane = lax.broadcasted_iota(jnp.int32, (R, C), 1)
    cin = (v >> (lane % 16)) & 1

    o_ref[...] = _brev8((c + cin) & 0xFF).astype(jnp.uint8)

    block_fold = agg[R - 1, 0]
    carry_ref[0] = jnp.where(block_fold == two, carry_in, block_fold)


@jax.jit
def kernel(a, b):
    n = a.shape[0]
    rows = n // LANES
    a2 = a.reshape(rows, LANES)
    b2 = b.reshape(rows, LANES)
    rpb = min(ROWS_PER_BLOCK, rows)
    grid = rows // rpb

    out = pl.pallas_call(
        _bsadd_block,
        grid=(grid,),
        in_specs=[
            pl.BlockSpec((rpb, LANES), lambda i: (i, 0)),
            pl.BlockSpec((rpb, LANES), lambda i: (i, 0)),
        ],
        out_specs=pl.BlockSpec((rpb, LANES), lambda i: (i, 0)),
        out_shape=jax.ShapeDtypeStruct((rows, LANES), jnp.uint8),
        scratch_shapes=[pltpu.SMEM((1,), jnp.int32)],
        compiler_params=pltpu.CompilerParams(
            dimension_semantics=("arbitrary",)),
    )(a2, b2)

    return out.reshape(n)


# speedup vs baseline: 1.6197x; 1.6197x over previous
import jax
import jax.numpy as jnp
import numpy as np
from jax import lax
from jax.experimental import pallas as pl
from jax.experimental.pallas import tpu as pltpu

LANES = 512
ROWS_PER_BLOCK = 512

_M7 = np.int32(0x7F7F7F7F)
_M8 = np.uint32(0x80808080)
_M1 = np.int32(0x01010101)
_B4 = np.int32(0x0F0F0F0F)
_B2 = np.int32(0x33333333)
_B1 = np.int32(0x55555555)


def _brev_swar(v):
    v = ((v & _B4) << 4) | ((v >> 4) & _B4)
    v = ((v & _B2) << 2) | ((v >> 2) & _B2)
    v = ((v & _B1) << 1) | ((v >> 1) & _B1)
    return v


def _pack16(bits_f32, n, h):
    ji = lax.broadcasted_iota(jnp.int32, (n, n // 16), 0)
    hi = lax.broadcasted_iota(jnp.int32, (n, n // 16), 1)
    w = jnp.where(ji // 16 == hi, 1 << (ji % 16), 0).astype(jnp.float32)
    return jnp.dot(bits_f32, w,
                   preferred_element_type=jnp.float32).astype(jnp.int32)


def _unpack16(carries, n):
    h = n // 16
    lo = carries & 0xFF
    hi = carries >> 8
    cat = jnp.concatenate([lo, hi], axis=1).astype(jnp.float32)
    ri = lax.broadcasted_iota(jnp.int32, (2 * h, n), 0)
    jj = lax.broadcasted_iota(jnp.int32, (2 * h, n), 1)
    sel = jnp.where(jj // 16 == ri % h,
                    jnp.where(ri < h, 1, 256), 0).astype(jnp.float32)
    v = jnp.dot(cat, sel,
                preferred_element_type=jnp.float32).astype(jnp.int32)
    lane = lax.broadcasted_iota(jnp.int32, v.shape, 1)
    return (v >> (lane % 16)) & 1


def _bsadd_block(a_ref, b_ref, o_ref, carry_ref):
    @pl.when(pl.program_id(0) == 0)
    def _():
        carry_ref[0] = 0

    a = _brev_swar(pltpu.bitcast(a_ref[...], jnp.int32))
    b = _brev_swar(pltpu.bitcast(b_ref[...], jnp.int32))

    axb = a ^ b
    s_low = (a & _M7) + (b & _M7)
    c = s_low ^ (axb & _M8.astype(jnp.int32))
    g_bits = ((a & b) | (axb & s_low)) & _M8.astype(jnp.int32)
    p_bits = ((c & _M7) + _M1) & c & _M8.astype(jnp.int32)

    W, C = c.shape
    two = jnp.int32(2)

    planes = [((g_bits >> (8 * t + 7)) & 1) for t in range(4)] + \
             [((p_bits >> (8 * t + 7)) & 1) for t in range(4)]
    stacked = jnp.concatenate(planes, axis=0).astype(jnp.float32)
    packed = _pack16(stacked, C, C // 16)
    gm_t = [packed[t * W:(t + 1) * W] for t in range(4)]
    pm_t = [packed[(4 + t) * W:(5 + t) * W] for t in range(4)]

    gh, ph = [], []
    for t in range(4):
        pg = pm_t[t] | gm_t[t]
        gh.append((gm_t[t] + pg) >> 16)
        ph.append((pm_t[t] == 0xFFFF).astype(jnp.int32))
    gh_all = jnp.concatenate(gh, axis=1)
    ph_all = jnp.concatenate(ph, axis=1)

    stacked2 = jnp.concatenate([gh_all, ph_all], axis=0).astype(jnp.float32)
    packed2 = _pack16(stacked2, 128, 8)
    gm2 = packed2[:W]
    pm2 = packed2[W:]
    pg2 = pm2 | gm2

    f3 = jnp.where(pm2 == 0xFFFF, 2, (gm2 + pg2) >> 16)
    f = f3
    k = 1
    while k < 8:
        shifted = jnp.concatenate(
            [jnp.full((W, k), two, jnp.int32), f[:, : 8 - k]], axis=1)
        f = jnp.where(f == two, shifted, f)
        k *= 2

    agg = f[:, 7:8]
    k = 1
    while k < W:
        shifted = jnp.concatenate(
            [jnp.full((k, 1), two, jnp.int32), agg[: W - k]], axis=0)
        agg = jnp.where(agg == two, shifted, agg)
        k *= 2
    carry_in = carry_ref[0]
    row_excl = jnp.concatenate(
        [jnp.full((1, 1), two, jnp.int32), agg[: W - 1]], axis=0)
    row_pref = jnp.where(row_excl == two, carry_in, row_excl)

    e = jnp.concatenate(
        [jnp.full((W, 1), two, jnp.int32), f[:, :7]], axis=1)
    ch3 = jnp.where(e == two, row_pref, e)

    carries2 = ((gm2 + pg2 + ch3) ^ gm2 ^ pg2) & 0xFFFF
    ch_all = _unpack16(carries2, 128)

    cin_word = jnp.zeros_like(c)
    for t in range(4):
        ch_t = ch_all[:, t * 32:(t + 1) * 32]
        pg = pm_t[t] | gm_t[t]
        car = ((gm_t[t] + pg + ch_t) ^ gm_t[t] ^ pg) & 0xFFFF
        cin_t = _unpack16(car, C)
        cin_word = cin_word | (cin_t << (8 * t))

    res = ((c & _M7) + cin_word) ^ (c & _M8.astype(jnp.int32))
    res = _brev_swar(res)
    o_ref[...] = pltpu.bitcast(res, jnp.uint8)

    block_fold = agg[W - 1, 0]
    carry_ref[0] = jnp.where(block_fold == two, carry_in, block_fold)


@jax.jit
def kernel(a, b):
    n = a.shape[0]
    rows = n // LANES
    a2 = a.reshape(rows, LANES)
    b2 = b.reshape(rows, LANES)
    rpb = min(ROWS_PER_BLOCK, rows)
    grid = rows // rpb

    out = pl.pallas_call(
        _bsadd_block,
        grid=(grid,),
        in_specs=[
            pl.BlockSpec((rpb, LANES), lambda i: (i, 0)),
            pl.BlockSpec((rpb, LANES), lambda i: (i, 0)),
        ],
        out_specs=pl.BlockSpec((rpb, LANES), lambda i: (i, 0)),
        out_shape=jax.ShapeDtypeStruct((rows, LANES), jnp.uint8),
        scratch_shapes=[pltpu.SMEM((1,), jnp.int32)],
        compiler_params=pltpu.CompilerParams(
            dimension_semantics=("arbitrary",)),
    )(a2, b2)

    return out.reshape(n)


# speedup vs baseline: 2.1511x; 1.3281x over previous
import jax
import jax.numpy as jnp
import numpy as np
from jax import lax
from jax.experimental import pallas as pl
from jax.experimental.pallas import tpu as pltpu

LANES = 512
ROWS_PER_BLOCK = 2048

_M7 = np.int32(0x7F7F7F7F)
_M8 = np.uint32(0x80808080)
_M1 = np.int32(0x01010101)
_B4 = np.int32(0x0F0F0F0F)
_B2 = np.int32(0x33333333)
_B1 = np.int32(0x55555555)


def _brev_swar(v):
    v = ((v & _B4) << 4) | ((v >> 4) & _B4)
    v = ((v & _B2) << 2) | ((v >> 2) & _B2)
    v = ((v & _B1) << 1) | ((v >> 1) & _B1)
    return v


def _pack16(bits_f32, n, h):
    ji = lax.broadcasted_iota(jnp.int32, (n, n // 16), 0)
    hi = lax.broadcasted_iota(jnp.int32, (n, n // 16), 1)
    w = jnp.where(ji // 16 == hi, 1 << (ji % 16), 0).astype(jnp.float32)
    return jnp.dot(bits_f32, w,
                   preferred_element_type=jnp.float32).astype(jnp.int32)


def _unpack16(carries, n):
    h = n // 16
    lo = carries & 0xFF
    hi = carries >> 8
    cat = jnp.concatenate([lo, hi], axis=1).astype(jnp.float32)
    ri = lax.broadcasted_iota(jnp.int32, (2 * h, n), 0)
    jj = lax.broadcasted_iota(jnp.int32, (2 * h, n), 1)
    sel = jnp.where(jj // 16 == ri % h,
                    jnp.where(ri < h, 1, 256), 0).astype(jnp.float32)
    v = jnp.dot(cat, sel,
                preferred_element_type=jnp.float32).astype(jnp.int32)
    lane = lax.broadcasted_iota(jnp.int32, v.shape, 1)
    return (v >> (lane % 16)) & 1


def _bsadd_block(a_ref, b_ref, o_ref, carry_ref):
    @pl.when(pl.program_id(0) == 0)
    def _():
        carry_ref[0] = 0

    a = _brev_swar(pltpu.bitcast(a_ref[...], jnp.int32))
    b = _brev_swar(pltpu.bitcast(b_ref[...], jnp.int32))

    axb = a ^ b
    s_low = (a & _M7) + (b & _M7)
    c = s_low ^ (axb & _M8.astype(jnp.int32))
    g_bits = ((a & b) | (axb & s_low)) & _M8.astype(jnp.int32)
    p_bits = ((c & _M7) + _M1) & c & _M8.astype(jnp.int32)

    W, C = c.shape
    two = jnp.int32(2)

    planes = [((g_bits >> (8 * t + 7)) & 1) for t in range(4)] + \
             [((p_bits >> (8 * t + 7)) & 1) for t in range(4)]
    stacked = jnp.concatenate(planes, axis=0).astype(jnp.float32)
    packed = _pack16(stacked, C, C // 16)
    gm_t = [packed[t * W:(t + 1) * W] for t in range(4)]
    pm_t = [packed[(4 + t) * W:(5 + t) * W] for t in range(4)]

    gh, ph = [], []
    for t in range(4):
        pg = pm_t[t] | gm_t[t]
        gh.append((gm_t[t] + pg) >> 16)
        ph.append((pm_t[t] == 0xFFFF).astype(jnp.int32))
    gh_all = jnp.concatenate(gh, axis=1)
    ph_all = jnp.concatenate(ph, axis=1)

    stacked2 = jnp.concatenate([gh_all, ph_all], axis=0).astype(jnp.float32)
    packed2 = _pack16(stacked2, 128, 8)
    gm2 = packed2[:W]
    pm2 = packed2[W:]
    pg2 = pm2 | gm2

    f3 = jnp.where(pm2 == 0xFFFF, 2, (gm2 + pg2) >> 16)
    f = f3
    k = 1
    while k < 8:
        shifted = jnp.concatenate(
            [jnp.full((W, k), two, jnp.int32), f[:, : 8 - k]], axis=1)
        f = jnp.where(f == two, shifted, f)
        k *= 2

    agg = f[:, 7:8]
    k = 1
    while k < W:
        shifted = jnp.concatenate(
            [jnp.full((k, 1), two, jnp.int32), agg[: W - k]], axis=0)
        agg = jnp.where(agg == two, shifted, agg)
        k *= 2
    carry_in = carry_ref[0]
    row_excl = jnp.concatenate(
        [jnp.full((1, 1), two, jnp.int32), agg[: W - 1]], axis=0)
    row_pref = jnp.where(row_excl == two, carry_in, row_excl)

    e = jnp.concatenate(
        [jnp.full((W, 1), two, jnp.int32), f[:, :7]], axis=1)
    ch3 = jnp.where(e == two, row_pref, e)

    carries2 = ((gm2 + pg2 + ch3) ^ gm2 ^ pg2) & 0xFFFF
    ch_all = _unpack16(carries2, 128)

    cin_word = jnp.zeros_like(c)
    for t in range(4):
        ch_t = ch_all[:, t * 32:(t + 1) * 32]
        pg = pm_t[t] | gm_t[t]
        car = ((gm_t[t] + pg + ch_t) ^ gm_t[t] ^ pg) & 0xFFFF
        cin_t = _unpack16(car, C)
        cin_word = cin_word | (cin_t << (8 * t))

    res = ((c & _M7) + cin_word) ^ (c & _M8.astype(jnp.int32))
    res = _brev_swar(res)
    o_ref[...] = pltpu.bitcast(res, jnp.uint8)

    block_fold = agg[W - 1, 0]
    carry_ref[0] = jnp.where(block_fold == two, carry_in, block_fold)


@jax.jit
def kernel(a, b):
    n = a.shape[0]
    rows = n // LANES
    a2 = a.reshape(rows, LANES)
    b2 = b.reshape(rows, LANES)
    rpb = min(ROWS_PER_BLOCK, rows)
    grid = rows // rpb

    out = pl.pallas_call(
        _bsadd_block,
        grid=(grid,),
        in_specs=[
            pl.BlockSpec((rpb, LANES), lambda i: (i, 0)),
            pl.BlockSpec((rpb, LANES), lambda i: (i, 0)),
        ],
        out_specs=pl.BlockSpec((rpb, LANES), lambda i: (i, 0)),
        out_shape=jax.ShapeDtypeStruct((rows, LANES), jnp.uint8),
        scratch_shapes=[pltpu.SMEM((1,), jnp.int32)],
        compiler_params=pltpu.CompilerParams(
            dimension_semantics=("arbitrary",)),
    )(a2, b2)

    return out.reshape(n)


# speedup vs baseline: 2.1862x; 1.0163x over previous
import jax
import jax.numpy as jnp
import numpy as np
from jax import lax
from jax.experimental import pallas as pl
from jax.experimental.pallas import tpu as pltpu

LANES = 512
ROWS_PER_BLOCK = 4096

_M7 = np.int32(0x7F7F7F7F)
_M8 = np.uint32(0x80808080)
_M1 = np.int32(0x01010101)
_B4 = np.int32(0x0F0F0F0F)
_B2 = np.int32(0x33333333)
_B1 = np.int32(0x55555555)


def _brev_swar(v):
    v = ((v & _B4) << 4) | ((v >> 4) & _B4)
    v = ((v & _B2) << 2) | ((v >> 2) & _B2)
    v = ((v & _B1) << 1) | ((v >> 1) & _B1)
    return v


def _pack16(bits_f32, n, h):
    ji = lax.broadcasted_iota(jnp.int32, (n, n // 16), 0)
    hi = lax.broadcasted_iota(jnp.int32, (n, n // 16), 1)
    w = jnp.where(ji // 16 == hi, 1 << (ji % 16), 0).astype(jnp.float32)
    return jnp.dot(bits_f32, w,
                   preferred_element_type=jnp.float32).astype(jnp.int32)


def _unpack16(carries, n):
    h = n // 16
    lo = carries & 0xFF
    hi = carries >> 8
    cat = jnp.concatenate([lo, hi], axis=1).astype(jnp.float32)
    ri = lax.broadcasted_iota(jnp.int32, (2 * h, n), 0)
    jj = lax.broadcasted_iota(jnp.int32, (2 * h, n), 1)
    sel = jnp.where(jj // 16 == ri % h,
                    jnp.where(ri < h, 1, 256), 0).astype(jnp.float32)
    v = jnp.dot(cat, sel,
                preferred_element_type=jnp.float32).astype(jnp.int32)
    lane = lax.broadcasted_iota(jnp.int32, v.shape, 1)
    return (v >> (lane % 16)) & 1


def _bsadd_block(a_ref, b_ref, o_ref, carry_ref):
    @pl.when(pl.program_id(0) == 0)
    def _():
        carry_ref[0] = 0

    a = _brev_swar(pltpu.bitcast(a_ref[...], jnp.int32))
    b = _brev_swar(pltpu.bitcast(b_ref[...], jnp.int32))

    axb = a ^ b
    s_low = (a & _M7) + (b & _M7)
    c = s_low ^ (axb & _M8.astype(jnp.int32))
    g_bits = ((a & b) | (axb & s_low)) & _M8.astype(jnp.int32)
    p_bits = ((c & _M7) + _M1) & c & _M8.astype(jnp.int32)

    W, C = c.shape
    two = jnp.int32(2)

    planes = [((g_bits >> (8 * t + 7)) & 1) for t in range(4)] + \
             [((p_bits >> (8 * t + 7)) & 1) for t in range(4)]
    stacked = jnp.concatenate(planes, axis=0).astype(jnp.float32)
    packed = _pack16(stacked, C, C // 16)
    gm_t = [packed[t * W:(t + 1) * W] for t in range(4)]
    pm_t = [packed[(4 + t) * W:(5 + t) * W] for t in range(4)]

    gh, ph = [], []
    for t in range(4):
        pg = pm_t[t] | gm_t[t]
        gh.append((gm_t[t] + pg) >> 16)
        ph.append((pm_t[t] == 0xFFFF).astype(jnp.int32))
    gh_all = jnp.concatenate(gh, axis=1)
    ph_all = jnp.concatenate(ph, axis=1)

    stacked2 = jnp.concatenate([gh_all, ph_all], axis=0).astype(jnp.float32)
    packed2 = _pack16(stacked2, 128, 8)
    gm2 = packed2[:W]
    pm2 = packed2[W:]
    pg2 = pm2 | gm2

    f3 = jnp.where(pm2 == 0xFFFF, 2, (gm2 + pg2) >> 16)
    f = f3
    k = 1
    while k < 8:
        shifted = jnp.concatenate(
            [jnp.full((W, k), two, jnp.int32), f[:, : 8 - k]], axis=1)
        f = jnp.where(f == two, shifted, f)
        k *= 2

    agg = f[:, 7:8]
    k = 1
    while k < W:
        shifted = jnp.concatenate(
            [jnp.full((k, 1), two, jnp.int32), agg[: W - k]], axis=0)
        agg = jnp.where(agg == two, shifted, agg)
        k *= 2
    carry_in = carry_ref[0]
    row_excl = jnp.concatenate(
        [jnp.full((1, 1), two, jnp.int32), agg[: W - 1]], axis=0)
    row_pref = jnp.where(row_excl == two, carry_in, row_excl)

    e = jnp.concatenate(
        [jnp.full((W, 1), two, jnp.int32), f[:, :7]], axis=1)
    ch3 = jnp.where(e == two, row_pref, e)

    carries2 = ((gm2 + pg2 + ch3) ^ gm2 ^ pg2) & 0xFFFF
    ch_all = _unpack16(carries2, 128)

    cin_word = jnp.zeros_like(c)
    for t in range(4):
        ch_t = ch_all[:, t * 32:(t + 1) * 32]
        pg = pm_t[t] | gm_t[t]
        car = ((gm_t[t] + pg + ch_t) ^ gm_t[t] ^ pg) & 0xFFFF
        cin_t = _unpack16(car, C)
        cin_word = cin_word | (cin_t << (8 * t))

    res = ((c & _M7) + cin_word) ^ (c & _M8.astype(jnp.int32))
    res = _brev_swar(res)
    o_ref[...] = pltpu.bitcast(res, jnp.uint8)

    block_fold = agg[W - 1, 0]
    carry_ref[0] = jnp.where(block_fold == two, carry_in, block_fold)


@jax.jit
def kernel(a, b):
    n = a.shape[0]
    rows = n // LANES
    a2 = a.reshape(rows, LANES)
    b2 = b.reshape(rows, LANES)
    rpb = min(ROWS_PER_BLOCK, rows)
    grid = rows // rpb

    out = pl.pallas_call(
        _bsadd_block,
        grid=(grid,),
        in_specs=[
            pl.BlockSpec((rpb, LANES), lambda i: (i, 0)),
            pl.BlockSpec((rpb, LANES), lambda i: (i, 0)),
        ],
        out_specs=pl.BlockSpec((rpb, LANES), lambda i: (i, 0)),
        out_shape=jax.ShapeDtypeStruct((rows, LANES), jnp.uint8),
        scratch_shapes=[pltpu.SMEM((1,), jnp.int32)],
        compiler_params=pltpu.CompilerParams(
            dimension_semantics=("arbitrary",)),
    )(a2, b2)

    return out.reshape(n)
